# Initial kernel scaffold; baseline (speedup 1.0000x reference)
import jax
import jax.numpy as jnp
from jax.experimental import pallas as pl


def kernel(x, edge_index, edge_attr, Wn, bn, We, be, W1, b1, W2, b2, gamma, beta):
    raise NotImplementedError("write your pallas kernel here")


# speedup vs baseline: 3.3474x; 3.3474x over previous
import functools

import jax
import jax.numpy as jnp
from jax import lax
from jax.experimental import pallas as pl
from jax.experimental.pallas import tpu as pltpu
from jax.experimental.pallas import tpu_sc as plsc

N = 50000
E = 800000
H = 64
HH = 32
LAYERS = 6
EPS_LN = 1e-5

NTILES = 16
K = 512
GPC = K // 128
CHUNKS = 98
EP = NTILES * CHUNKS * K
G = EP // 128
AGG_ROWS = 50048
ROWS_PER_TILE = AGG_ROWS // NTILES
ZROWS = 136
ZCOPIES = ROWS_PER_TILE // ZROWS

BN = 2000
NB = N // BN
BE = 8192
CNT = float(N * H)


def _sc_edge_body(h2, src2, dstp, e2, agg_out, src_buf, dst_buf, msg_buf,
                  zbuf, agg_spmem, sem):
    cid = lax.axis_index("c")
    sid = lax.axis_index("s")

    def _zero_row(r, carry):
        zbuf[r, pl.ds(0, 16)] = jnp.zeros((16,), jnp.float32)
        zbuf[r, pl.ds(16, 16)] = jnp.zeros((16,), jnp.float32)
        return carry
    lax.fori_loop(0, ZROWS, _zero_row, 0)

    def _zero_copy(kk, carry):
        pltpu.sync_copy(
            zbuf, agg_spmem.at[pl.ds(sid * ROWS_PER_TILE + kk * ZROWS, ZROWS)])
        return carry
    lax.fori_loop(0, ZCOPIES, _zero_copy, 0)
    plsc.subcore_barrier()

    def _chunk(t, carry):
        ch = sid * CHUNKS + t
        gbase = ch * GPC
        ebase = ch * K
        pltpu.sync_copy(src2.at[cid, pl.ds(gbase, GPC)], src_buf)
        pltpu.sync_copy(dstp.at[pl.ds(gbase, GPC)], dst_buf)
        pltpu.sync_copy(e2.at[cid, pl.ds(ebase, K)], msg_buf)
        descs = [
            pltpu.async_copy(h2.at[src_buf.at[j]],
                             msg_buf.at[pl.ds(j * 128, 128)], sem, add=True)
            for j in range(GPC)
        ]
        for d in descs:
            d.wait()

        def _relu(r, carry2):
            for rr in range(8):
                for q in (0, 16):
                    v = msg_buf[r * 8 + rr, pl.ds(q, 16)]
                    msg_buf[r * 8 + rr, pl.ds(q, 16)] = jnp.maximum(v, 0.0)
            return carry2
        lax.fori_loop(0, K // 8, _relu, 0)

        for j in range(GPC):
            pltpu.sync_copy(msg_buf.at[pl.ds(j * 128, 128)],
                            agg_spmem.at[dst_buf.at[j]], add=True)
        return carry
    lax.fori_loop(0, CHUNKS, _chunk, 0)
    plsc.subcore_barrier()

    pltpu.sync_copy(
        agg_spmem.at[pl.ds(sid * ROWS_PER_TILE, ROWS_PER_TILE)],
        agg_out.at[cid, pl.ds(sid * ROWS_PER_TILE, ROWS_PER_TILE)])


@functools.lru_cache(maxsize=None)
def _get_sc_edge():
    return pl.kernel(
        _sc_edge_body,
        out_type=jax.ShapeDtypeStruct((2, AGG_ROWS, HH), jnp.float32),
        mesh=plsc.VectorSubcoreMesh(core_axis_name="c", subcore_axis_name="s"),
        scratch_types=[
            pltpu.VMEM((GPC, 128), jnp.int32),
            pltpu.VMEM((GPC, 128), jnp.int32),
            pltpu.VMEM((K, HH), jnp.float32),
            pltpu.VMEM((ZROWS, HH), jnp.float32),
            pltpu.VMEM_SHARED((AGG_ROWS, HH), jnp.float32),
            pltpu.SemaphoreType.DMA,
        ],
        compiler_params=pltpu.CompilerParams(use_tc_tiling_on_sc=False),
    )


def _proj_nodes_body(x_ref, wn_ref, bn_ref, hs_ref):
    h = x_ref[...] * wn_ref[...] + bn_ref[...]
    hs_ref[0] = h[:, :HH]
    hs_ref[1] = h[:, HH:]


def _proj_edges_body(ea_ref, we_ref, be_ref, e2_ref):
    ea = ea_ref[...]
    e = (ea[:, 0:1] * we_ref[0:1, :] + ea[:, 1:2] * we_ref[1:2, :]
         + be_ref[...])
    e2_ref[0] = e[:, :HH]
    e2_ref[1] = e[:, HH:]


def _mlp_body(hs_ref, agg_ref, w1_ref, b1_ref, w2_ref, b2_ref,
              z_ref, ssum_ref, ssq_ref):
    h = jnp.concatenate([hs_ref[0], hs_ref[1]], axis=1)
    a = jnp.concatenate([agg_ref[0], agg_ref[1]], axis=1)
    zin = h + a
    t = jnp.maximum(
        jnp.dot(zin, w1_ref[...], preferred_element_type=jnp.float32,
                precision=lax.Precision.HIGHEST) + b1_ref[...], 0.0)
    z = jnp.dot(t, w2_ref[...], preferred_element_type=jnp.float32,
                precision=lax.Precision.HIGHEST) + b2_ref[...]
    z_ref[...] = z

    @pl.when(pl.program_id(0) == 0)
    def _():
        ssum_ref[0, 0] = 0.0
        ssq_ref[0, 0] = 0.0

    ssum_ref[0, 0] += jnp.sum(z)
    ssq_ref[0, 0] += jnp.sum(z * z)


def _norm_body(z_ref, hs_ref, ssum_ref, ssq_ref, g_ref, b_ref, out_ref,
               *, full):
    mu = ssum_ref[0, 0] / CNT
    var = jnp.maximum(ssq_ref[0, 0] / CNT - mu * mu, 0.0)
    rstd = 1.0 / (jnp.sqrt(var) + EPS_LN)
    z = (z_ref[...] - mu) * rstd * g_ref[...] + b_ref[...]
    z = jnp.maximum(z, 0.0)
    hn = z + jnp.concatenate([hs_ref[0], hs_ref[1]], axis=1)
    if full:
        out_ref[...] = hn
    else:
        out_ref[0] = hn[:, :HH]
        out_ref[1] = hn[:, HH:]


_half_spec = pl.BlockSpec((2, BN, HH), lambda i: (0, i, 0))
_full_spec = pl.BlockSpec((BN, H), lambda i: (i, 0))
_scal_spec = pl.BlockSpec((1, 1), lambda i: (0, 0), memory_space=pltpu.SMEM)


def _wfull(shape):
    return pl.BlockSpec(shape, lambda i: tuple(0 for _ in shape))


_proj_nodes = pl.pallas_call(
    _proj_nodes_body,
    grid=(NB,),
    in_specs=[pl.BlockSpec((BN, 1), lambda i: (i, 0)),
              _wfull((1, H)), _wfull((1, H))],
    out_specs=_half_spec,
    out_shape=jax.ShapeDtypeStruct((2, N, HH), jnp.float32),
)

_proj_edges = pl.pallas_call(
    _proj_edges_body,
    grid=(EP // BE,),
    in_specs=[pl.BlockSpec((BE, 2), lambda i: (i, 0)),
              _wfull((2, H)), _wfull((1, H))],
    out_specs=pl.BlockSpec((2, BE, HH), lambda i: (0, i, 0)),
    out_shape=jax.ShapeDtypeStruct((2, EP, HH), jnp.float32),
)

_mlp = pl.pallas_call(
    _mlp_body,
    grid=(NB,),
    in_specs=[_half_spec, _half_spec,
              _wfull((H, H)), _wfull((1, H)), _wfull((H, H)), _wfull((1, H))],
    out_specs=(_full_spec, _scal_spec, _scal_spec),
    out_shape=(jax.ShapeDtypeStruct((N, H), jnp.float32),
               jax.ShapeDtypeStruct((1, 1), jnp.float32),
               jax.ShapeDtypeStruct((1, 1), jnp.float32)),
)

_norm_half = pl.pallas_call(
    functools.partial(_norm_body, full=False),
    grid=(NB,),
    in_specs=[_full_spec, _half_spec, _scal_spec, _scal_spec,
              _wfull((1, H)), _wfull((1, H))],
    out_specs=_half_spec,
    out_shape=jax.ShapeDtypeStruct((2, N, HH), jnp.float32),
)

_norm_full = pl.pallas_call(
    functools.partial(_norm_body, full=True),
    grid=(NB,),
    in_specs=[_full_spec, _half_spec, _scal_spec, _scal_spec,
              _wfull((1, H)), _wfull((1, H))],
    out_specs=_full_spec,
    out_shape=jax.ShapeDtypeStruct((N, H), jnp.float32),
)


def kernel(x, edge_index, edge_attr, Wn, bn, We, be, W1, b1, W2, b2,
           gamma, beta):
    src = edge_index[0].astype(jnp.int32)
    dst = edge_index[1].astype(jnp.int32)

    pad = EP - E
    src2 = jnp.stack([src, src + N])
    src2 = jnp.pad(src2, ((0, 0), (0, pad))).reshape(2, G, 128)
    dstp = jnp.pad(dst, (0, pad), constant_values=N).reshape(G, 128)
    ea_pad = jnp.pad(edge_attr, ((0, pad), (0, 0)))

    hs = _proj_nodes(x, Wn.reshape(1, H), bn.reshape(1, H))
    e2 = _proj_edges(ea_pad, We, be.reshape(1, H))

    for l in range(LAYERS):
        h2 = hs.reshape(2 * N, HH)
        agg = _get_sc_edge()(h2, src2, dstp, e2)
        z, ssum, ssq = _mlp(hs, agg, W1[l], b1[l].reshape(1, H),
                            W2[l], b2[l].reshape(1, H))
        gl = gamma[l].reshape(1, H)
        bl = beta[l].reshape(1, H)
        if l < LAYERS - 1:
            hs = _norm_half(z, hs, ssum, ssq, gl, bl)
        else:
            out = _norm_full(z, hs, ssum, ssq, gl, bl)
    return out


# speedup vs baseline: 5.0222x; 1.5003x over previous
import functools

import jax
import jax.numpy as jnp
from jax import lax
from jax.experimental import pallas as pl
from jax.experimental.pallas import tpu as pltpu
from jax.experimental.pallas import tpu_sc as plsc

N = 50000
E = 800000
H = 64
HH = 32
LAYERS = 6
EPS_LN = 1e-5

NTILES = 16
K = 256
GPC = K // 128
CHUNKS = 196
EP = NTILES * CHUNKS * K
NCH = NTILES * CHUNKS
AGG_ROWS = 50048
ROWS_PER_TILE = AGG_ROWS // NTILES

BN = 2000
NB = N // BN
BE = 8192
CNT = float(N * H)


def _sc_edge_body(h2, idxpack, eapack, weh, beh, agg_out,
                  idx0, idx1, ea0, ea1, msg0, msg1, wbuf,
                  agg_spmem, sg0, sg1, ss0, ss1):
    cid = lax.axis_index("c")
    sid = lax.axis_index("s")
    idx_bufs = (idx0, idx1)
    ea_bufs = (ea0, ea1)
    msg_bufs = (msg0, msg1)
    sgs = (sg0, sg1)
    sss = (ss0, ss1)

    pltpu.sync_copy(weh.at[0, cid], wbuf.at[0])
    pltpu.sync_copy(weh.at[1, cid], wbuf.at[1])
    pltpu.sync_copy(beh.at[cid], wbuf.at[2])
    w0a = wbuf[0, pl.ds(0, 16)]
    w0b = wbuf[0, pl.ds(16, 16)]
    w1a = wbuf[1, pl.ds(0, 16)]
    w1b = wbuf[1, pl.ds(16, 16)]
    bba = wbuf[2, pl.ds(0, 16)]
    bbb = wbuf[2, pl.ds(16, 16)]

    def _zero_row(r, carry):
        for rr in range(8):
            msg0[r * 8 + rr, pl.ds(0, 16)] = jnp.zeros((16,), jnp.float32)
            msg0[r * 8 + rr, pl.ds(16, 16)] = jnp.zeros((16,), jnp.float32)
        return carry
    lax.fori_loop(0, K // 8, _zero_row, 0)

    def _zero_copy(kk, carry):
        pltpu.sync_copy(
            msg0, agg_spmem.at[pl.ds(sid * ROWS_PER_TILE + kk * K, K)])
        return carry
    lax.fori_loop(0, ROWS_PER_TILE // K, _zero_copy, 0)
    pltpu.sync_copy(
        msg0.at[pl.ds(0, ROWS_PER_TILE % K)],
        agg_spmem.at[pl.ds(sid * ROWS_PER_TILE + (ROWS_PER_TILE // K) * K,
                           ROWS_PER_TILE % K)])
    plsc.subcore_barrier()

    def _g(t, b):
        ch = sid * CHUNKS + t
        mb, ib, eb = msg_bufs[b], idx_bufs[b], ea_bufs[b]

        @pl.when(t >= 2)
        def _():
            for j in range(GPC):
                pltpu.make_async_copy(
                    mb.at[pl.ds(j * 128, 128)], agg_spmem.at[ib.at[1, j]],
                    sss[b]).wait()

        pltpu.sync_copy(idxpack.at[cid, ch], ib)
        pltpu.sync_copy(eapack.at[ch], eb)

        def _ecomp(i, carry):
            va0 = eb[0, pl.ds(i * 16, 16)]
            va1 = eb[1, pl.ds(i * 16, 16)]
            for ii in range(16):
                a0 = va0[ii]
                a1 = va1[ii]
                mb[i * 16 + ii, pl.ds(0, 16)] = a0 * w0a + a1 * w1a + bba
                mb[i * 16 + ii, pl.ds(16, 16)] = a0 * w0b + a1 * w1b + bbb
            return carry
        lax.fori_loop(0, K // 16, _ecomp, 0)
        for j in range(GPC):
            pltpu.async_copy(h2.at[ib.at[0, j]],
                             mb.at[pl.ds(j * 128, 128)], sgs[b], add=True)

    def _r(t, b):
        mb, ib = msg_bufs[b], idx_bufs[b]
        for j in range(GPC):
            pltpu.make_async_copy(
                h2.at[ib.at[0, j]], mb.at[pl.ds(j * 128, 128)],
                sgs[b]).wait()

        def _relu(r, carry):
            for rr in range(8):
                for q in (0, 16):
                    v = mb[r * 8 + rr, pl.ds(q, 16)]
                    mb[r * 8 + rr, pl.ds(q, 16)] = jnp.maximum(v, 0.0)
            return carry
        lax.fori_loop(0, K // 8, _relu, 0)
        for j in range(GPC):
            pltpu.async_copy(mb.at[pl.ds(j * 128, 128)],
                             agg_spmem.at[ib.at[1, j]], sss[b], add=True)

    def _pair(g, carry):
        t0 = 2 * g
        _g(t0, 0)
        _g(t0 + 1, 1)
        _r(t0, 0)
        _r(t0 + 1, 1)
        return carry
    lax.fori_loop(0, CHUNKS // 2, _pair, 0)
    for b in range(2):
        for j in range(GPC):
            pltpu.make_async_copy(
                msg_bufs[b].at[pl.ds(j * 128, 128)],
                agg_spmem.at[idx_bufs[b].at[1, j]], sss[b]).wait()
    plsc.subcore_barrier()

    pltpu.sync_copy(
        agg_spmem.at[pl.ds(sid * ROWS_PER_TILE, ROWS_PER_TILE)],
        agg_out.at[cid, pl.ds(sid * ROWS_PER_TILE, ROWS_PER_TILE)])


@functools.lru_cache(maxsize=None)
def _get_sc_edge():
    return pl.kernel(
        _sc_edge_body,
        out_type=jax.ShapeDtypeStruct((2, AGG_ROWS, HH), jnp.float32),
        mesh=plsc.VectorSubcoreMesh(core_axis_name="c", subcore_axis_name="s"),
        scratch_types=[
            pltpu.VMEM((2, GPC, 128), jnp.int32),
            pltpu.VMEM((2, GPC, 128), jnp.int32),
            pltpu.VMEM((2, K), jnp.float32),
            pltpu.VMEM((2, K), jnp.float32),
            pltpu.VMEM((K, HH), jnp.float32),
            pltpu.VMEM((K, HH), jnp.float32),
            pltpu.VMEM((3, HH), jnp.float32),
            pltpu.VMEM_SHARED((AGG_ROWS, HH), jnp.float32),
            pltpu.SemaphoreType.DMA,
            pltpu.SemaphoreType.DMA,
            pltpu.SemaphoreType.DMA,
            pltpu.SemaphoreType.DMA,
        ],
        compiler_params=pltpu.CompilerParams(use_tc_tiling_on_sc=False),
    )


def _proj_nodes_body(x_ref, wn_ref, bn_ref, hs_ref):
    h = x_ref[...] * wn_ref[...] + bn_ref[...]
    hs_ref[0] = h[:, :HH]
    hs_ref[1] = h[:, HH:]


def _mlp_body(hs_ref, agg_ref, w1_ref, b1_ref, w2_ref, b2_ref,
              z_ref, ssum_ref, ssq_ref):
    h = jnp.concatenate([hs_ref[0], hs_ref[1]], axis=1)
    a = jnp.concatenate([agg_ref[0], agg_ref[1]], axis=1)
    zin = h + a
    t = jnp.maximum(
        jnp.dot(zin, w1_ref[...], preferred_element_type=jnp.float32,
                precision=lax.Precision.HIGHEST) + b1_ref[...], 0.0)
    z = jnp.dot(t, w2_ref[...], preferred_element_type=jnp.float32,
                precision=lax.Precision.HIGHEST) + b2_ref[...]
    z_ref[...] = z

    @pl.when(pl.program_id(0) == 0)
    def _():
        ssum_ref[0, 0] = 0.0
        ssq_ref[0, 0] = 0.0

    ssum_ref[0, 0] += jnp.sum(z)
    ssq_ref[0, 0] += jnp.sum(z * z)


def _norm_body(z_ref, hs_ref, ssum_ref, ssq_ref, g_ref, b_ref, out_ref,
               *, full):
    mu = ssum_ref[0, 0] / CNT
    var = jnp.maximum(ssq_ref[0, 0] / CNT - mu * mu, 0.0)
    rstd = 1.0 / (jnp.sqrt(var) + EPS_LN)
    z = (z_ref[...] - mu) * rstd * g_ref[...] + b_ref[...]
    z = jnp.maximum(z, 0.0)
    hn = z + jnp.concatenate([hs_ref[0], hs_ref[1]], axis=1)
    if full:
        out_ref[...] = hn
    else:
        out_ref[0] = hn[:, :HH]
        out_ref[1] = hn[:, HH:]


_half_spec = pl.BlockSpec((2, BN, HH), lambda i: (0, i, 0))
_full_spec = pl.BlockSpec((BN, H), lambda i: (i, 0))
_scal_spec = pl.BlockSpec((1, 1), lambda i: (0, 0), memory_space=pltpu.SMEM)


def _wfull(shape):
    return pl.BlockSpec(shape, lambda i: tuple(0 for _ in shape))


_proj_nodes = pl.pallas_call(
    _proj_nodes_body,
    grid=(NB,),
    in_specs=[pl.BlockSpec((BN, 1), lambda i: (i, 0)),
              _wfull((1, H)), _wfull((1, H))],
    out_specs=_half_spec,
    out_shape=jax.ShapeDtypeStruct((2, N, HH), jnp.float32),
)

_mlp = pl.pallas_call(
    _mlp_body,
    grid=(NB,),
    in_specs=[_half_spec, _half_spec,
              _wfull((H, H)), _wfull((1, H)), _wfull((H, H)), _wfull((1, H))],
    out_specs=(_full_spec, _scal_spec, _scal_spec),
    out_shape=(jax.ShapeDtypeStruct((N, H), jnp.float32),
               jax.ShapeDtypeStruct((1, 1), jnp.float32),
               jax.ShapeDtypeStruct((1, 1), jnp.float32)),
)

_norm_half = pl.pallas_call(
    functools.partial(_norm_body, full=False),
    grid=(NB,),
    in_specs=[_full_spec, _half_spec, _scal_spec, _scal_spec,
              _wfull((1, H)), _wfull((1, H))],
    out_specs=_half_spec,
    out_shape=jax.ShapeDtypeStruct((2, N, HH), jnp.float32),
)

_norm_full = pl.pallas_call(
    functools.partial(_norm_body, full=True),
    grid=(NB,),
    in_specs=[_full_spec, _half_spec, _scal_spec, _scal_spec,
              _wfull((1, H)), _wfull((1, H))],
    out_specs=_full_spec,
    out_shape=jax.ShapeDtypeStruct((N, H), jnp.float32),
)


def kernel(x, edge_index, edge_attr, Wn, bn, We, be, W1, b1, W2, b2,
           gamma, beta):
    src = edge_index[0].astype(jnp.int32)
    dst = edge_index[1].astype(jnp.int32)

    pad = EP - E
    src_p = jnp.pad(src, (0, pad))
    dst_p = jnp.pad(dst, (0, pad), constant_values=N)
    src2 = jnp.stack([src_p, src_p + N])
    idxpack = jnp.stack(
        [src2.reshape(2, NCH, GPC, 128),
         jnp.broadcast_to(dst_p.reshape(NCH, GPC, 128), (2, NCH, GPC, 128))],
        axis=2)
    eapack = jnp.pad(edge_attr, ((0, pad), (0, 0))).T.reshape(
        2, NCH, K).transpose(1, 0, 2)
    weh = We.reshape(2, 2, HH)
    beh = be.reshape(2, HH)

    hs = _proj_nodes(x, Wn.reshape(1, H), bn.reshape(1, H))

    for l in range(LAYERS):
        h2 = hs.reshape(2 * N, HH)
        agg = _get_sc_edge()(h2, idxpack, eapack, weh, beh)
        z, ssum, ssq = _mlp(hs, agg, W1[l], b1[l].reshape(1, H),
                            W2[l], b2[l].reshape(1, H))
        gl = gamma[l].reshape(1, H)
        bl = beta[l].reshape(1, H)
        if l < LAYERS - 1:
            hs = _norm_half(z, hs, ssum, ssq, gl, bl)
        else:
            out = _norm_full(z, hs, ssum, ssq, gl, bl)
    return out


# speedup vs baseline: 5.5863x; 1.1123x over previous
import functools

import jax
import jax.numpy as jnp
from jax import lax
from jax.experimental import pallas as pl
from jax.experimental.pallas import tpu as pltpu
from jax.experimental.pallas import tpu_sc as plsc

N = 50000
E = 800000
H = 64
HH = 32
LAYERS = 6
EPS_LN = 1e-5

NTILES = 16
K = 256
GPC = K // 128
CHUNKS = 196
EP = NTILES * CHUNKS * K
NCH = NTILES * CHUNKS
AGG_ROWS = 50048
ROWS_PER_TILE = AGG_ROWS // NTILES

BN = 2000
NB = N // BN
BE = 8192
CNT = float(N * H)


def _sc_edge_body(h2, pack, weh, beh, agg_out,
                  pk0, pk1, msg0, msg1, wbuf,
                  agg_spmem, sg0, sg1, ss0, ss1):
    cid = lax.axis_index("c")
    sid = lax.axis_index("s")
    idx_bufs = (pk0, pk1)
    msg_bufs = (msg0, msg1)
    sgs = (sg0, sg1)
    sss = (ss0, ss1)

    pltpu.sync_copy(weh.at[0, cid], wbuf.at[0])
    pltpu.sync_copy(weh.at[1, cid], wbuf.at[1])
    pltpu.sync_copy(beh.at[cid], wbuf.at[2])
    w0a = wbuf[0, pl.ds(0, 16)]
    w0b = wbuf[0, pl.ds(16, 16)]
    w1a = wbuf[1, pl.ds(0, 16)]
    w1b = wbuf[1, pl.ds(16, 16)]
    bba = wbuf[2, pl.ds(0, 16)]
    bbb = wbuf[2, pl.ds(16, 16)]

    def _zero_row(r, carry):
        for rr in range(8):
            msg0[r * 8 + rr, pl.ds(0, 16)] = jnp.zeros((16,), jnp.float32)
            msg0[r * 8 + rr, pl.ds(16, 16)] = jnp.zeros((16,), jnp.float32)
        return carry
    lax.fori_loop(0, K // 8, _zero_row, 0)

    def _zero_copy(kk, carry):
        pltpu.sync_copy(
            msg0, agg_spmem.at[pl.ds(sid * ROWS_PER_TILE + kk * K, K)])
        return carry
    lax.fori_loop(0, ROWS_PER_TILE // K, _zero_copy, 0)
    pltpu.sync_copy(
        msg0.at[pl.ds(0, ROWS_PER_TILE % K)],
        agg_spmem.at[pl.ds(sid * ROWS_PER_TILE + (ROWS_PER_TILE // K) * K,
                           ROWS_PER_TILE % K)])
    plsc.subcore_barrier()

    def _g(t, b):
        ch = sid * CHUNKS + t
        mb, ib = msg_bufs[b], idx_bufs[b]

        @pl.when(t >= 2)
        def _():
            for j in range(GPC):
                pltpu.make_async_copy(
                    mb.at[pl.ds(j * 128, 128)], agg_spmem.at[ib.at[1, j]],
                    sss[b]).wait()

        pltpu.sync_copy(pack.at[cid, ch], ib)

        def _ecomp(i, carry):
            g = i // 8
            q = (i % 8) * 16
            va0 = plsc.bitcast(ib[2, g, pl.ds(q, 16)], jnp.float32)
            va1 = plsc.bitcast(ib[3, g, pl.ds(q, 16)], jnp.float32)
            for ii in range(16):
                a0 = jnp.broadcast_to(va0[ii], (16,))
                a1 = jnp.broadcast_to(va1[ii], (16,))
                mb[i * 16 + ii, pl.ds(0, 16)] = a0 * w0a + a1 * w1a + bba
                mb[i * 16 + ii, pl.ds(16, 16)] = a0 * w0b + a1 * w1b + bbb
            return carry
        lax.fori_loop(0, K // 16, _ecomp, 0)
        for j in range(GPC):
            pltpu.async_copy(h2.at[ib.at[0, j]],
                             mb.at[pl.ds(j * 128, 128)], sgs[b], add=True)

    def _r(t, b):
        mb, ib = msg_bufs[b], idx_bufs[b]
        for j in range(GPC):
            pltpu.make_async_copy(
                h2.at[ib.at[0, j]], mb.at[pl.ds(j * 128, 128)],
                sgs[b]).wait()

        def _relu(r, carry):
            for rr in range(8):
                for q in (0, 16):
                    v = mb[r * 8 + rr, pl.ds(q, 16)]
                    mb[r * 8 + rr, pl.ds(q, 16)] = jnp.maximum(v, 0.0)
            return carry
        lax.fori_loop(0, K // 8, _relu, 0)
        for j in range(GPC):
            pltpu.async_copy(mb.at[pl.ds(j * 128, 128)],
                             agg_spmem.at[ib.at[1, j]], sss[b], add=True)

    def _pair(g, carry):
        t0 = 2 * g
        _g(t0, 0)
        _g(t0 + 1, 1)
        _r(t0, 0)
        _r(t0 + 1, 1)
        return carry
    lax.fori_loop(0, CHUNKS // 2, _pair, 0)
    for b in range(2):
        for j in range(GPC):
            pltpu.make_async_copy(
                msg_bufs[b].at[pl.ds(j * 128, 128)],
                agg_spmem.at[idx_bufs[b].at[1, j]], sss[b]).wait()
    plsc.subcore_barrier()

    pltpu.sync_copy(
        agg_spmem.at[pl.ds(sid * ROWS_PER_TILE, ROWS_PER_TILE)],
        agg_out.at[cid, pl.ds(sid * ROWS_PER_TILE, ROWS_PER_TILE)])


@functools.lru_cache(maxsize=None)
def _get_sc_edge():
    return pl.kernel(
        _sc_edge_body,
        out_type=jax.ShapeDtypeStruct((2, AGG_ROWS, HH), jnp.float32),
        mesh=plsc.VectorSubcoreMesh(core_axis_name="c", subcore_axis_name="s"),
        scratch_types=[
            pltpu.VMEM((4, GPC, 128), jnp.int32),
            pltpu.VMEM((4, GPC, 128), jnp.int32),
            pltpu.VMEM((K, HH), jnp.float32),
            pltpu.VMEM((K, HH), jnp.float32),
            pltpu.VMEM((3, HH), jnp.float32),
            pltpu.VMEM_SHARED((AGG_ROWS, HH), jnp.float32),
            pltpu.SemaphoreType.DMA,
            pltpu.SemaphoreType.DMA,
            pltpu.SemaphoreType.DMA,
            pltpu.SemaphoreType.DMA,
        ],
        compiler_params=pltpu.CompilerParams(use_tc_tiling_on_sc=False,
                                             needs_layout_passes=False),
    )


def _proj_nodes_body(x_ref, wn_ref, bn_ref, hs_ref):
    h = x_ref[...] * wn_ref[...] + bn_ref[...]
    hs_ref[0] = h[:, :HH]
    hs_ref[1] = h[:, HH:]


def _mlp_body(hs_ref, agg_ref, w1_ref, b1_ref, w2_ref, b2_ref,
              z_ref, ssum_ref, ssq_ref):
    h = jnp.concatenate([hs_ref[0], hs_ref[1]], axis=1)
    a = jnp.concatenate([agg_ref[0], agg_ref[1]], axis=1)
    zin = h + a
    t = jnp.maximum(
        jnp.dot(zin, w1_ref[...], preferred_element_type=jnp.float32,
                precision=lax.Precision.HIGHEST) + b1_ref[...], 0.0)
    z = jnp.dot(t, w2_ref[...], preferred_element_type=jnp.float32,
                precision=lax.Precision.HIGHEST) + b2_ref[...]
    z_ref[...] = z

    @pl.when(pl.program_id(0) == 0)
    def _():
        ssum_ref[0, 0] = 0.0
        ssq_ref[0, 0] = 0.0

    ssum_ref[0, 0] += jnp.sum(z)
    ssq_ref[0, 0] += jnp.sum(z * z)


def _norm_body(z_ref, hs_ref, ssum_ref, ssq_ref, g_ref, b_ref, out_ref,
               *, full):
    mu = ssum_ref[0, 0] / CNT
    var = jnp.maximum(ssq_ref[0, 0] / CNT - mu * mu, 0.0)
    rstd = 1.0 / (jnp.sqrt(var) + EPS_LN)
    z = (z_ref[...] - mu) * rstd * g_ref[...] + b_ref[...]
    z = jnp.maximum(z, 0.0)
    hn = z + jnp.concatenate([hs_ref[0], hs_ref[1]], axis=1)
    if full:
        out_ref[...] = hn
    else:
        out_ref[0] = hn[:, :HH]
        out_ref[1] = hn[:, HH:]


_half_spec = pl.BlockSpec((2, BN, HH), lambda i: (0, i, 0))
_full_spec = pl.BlockSpec((BN, H), lambda i: (i, 0))
_scal_spec = pl.BlockSpec((1, 1), lambda i: (0, 0), memory_space=pltpu.SMEM)


def _wfull(shape):
    return pl.BlockSpec(shape, lambda i: tuple(0 for _ in shape))


_proj_nodes = pl.pallas_call(
    _proj_nodes_body,
    grid=(NB,),
    in_specs=[pl.BlockSpec((BN, 1), lambda i: (i, 0)),
              _wfull((1, H)), _wfull((1, H))],
    out_specs=_half_spec,
    out_shape=jax.ShapeDtypeStruct((2, N, HH), jnp.float32),
)

_mlp = pl.pallas_call(
    _mlp_body,
    grid=(NB,),
    in_specs=[_half_spec, _half_spec,
              _wfull((H, H)), _wfull((1, H)), _wfull((H, H)), _wfull((1, H))],
    out_specs=(_full_spec, _scal_spec, _scal_spec),
    out_shape=(jax.ShapeDtypeStruct((N, H), jnp.float32),
               jax.ShapeDtypeStruct((1, 1), jnp.float32),
               jax.ShapeDtypeStruct((1, 1), jnp.float32)),
)

_norm_half = pl.pallas_call(
    functools.partial(_norm_body, full=False),
    grid=(NB,),
    in_specs=[_full_spec, _half_spec, _scal_spec, _scal_spec,
              _wfull((1, H)), _wfull((1, H))],
    out_specs=_half_spec,
    out_shape=jax.ShapeDtypeStruct((2, N, HH), jnp.float32),
)

_norm_full = pl.pallas_call(
    functools.partial(_norm_body, full=True),
    grid=(NB,),
    in_specs=[_full_spec, _half_spec, _scal_spec, _scal_spec,
              _wfull((1, H)), _wfull((1, H))],
    out_specs=_full_spec,
    out_shape=jax.ShapeDtypeStruct((N, H), jnp.float32),
)


def kernel(x, edge_index, edge_attr, Wn, bn, We, be, W1, b1, W2, b2,
           gamma, beta):
    src = edge_index[0].astype(jnp.int32)
    dst = edge_index[1].astype(jnp.int32)

    pad = EP - E
    src_p = jnp.pad(src, (0, pad))
    dst_p = jnp.pad(dst, (0, pad), constant_values=N)
    src2 = jnp.stack([src_p, src_p + N])
    ea_p = jnp.pad(edge_attr, ((0, pad), (0, 0)))
    eab = lax.bitcast_convert_type(ea_p.T, jnp.int32)
    rest = jnp.broadcast_to(
        jnp.stack([dst_p, eab[0], eab[1]]), (2, 3, EP))
    pack = jnp.concatenate([src2[:, None], rest], axis=1)
    pack = pack.reshape(2, 4, NCH, GPC, 128).transpose(0, 2, 1, 3, 4)
    weh = We.reshape(2, 2, HH)
    beh = be.reshape(2, HH)

    hs = _proj_nodes(x, Wn.reshape(1, H), bn.reshape(1, H))

    for l in range(LAYERS):
        h2 = hs.reshape(2 * N, HH)
        agg = _get_sc_edge()(h2, pack, weh, beh)
        z, ssum, ssq = _mlp(hs, agg, W1[l], b1[l].reshape(1, H),
                            W2[l], b2[l].reshape(1, H))
        gl = gamma[l].reshape(1, H)
        bl = beta[l].reshape(1, H)
        if l < LAYERS - 1:
            hs = _norm_half(z, hs, ssum, ssq, gl, bl)
        else:
            out = _norm_full(z, hs, ssum, ssq, gl, bl)
    return out


# speedup vs baseline: 6.3540x; 1.1374x over previous
import functools

import jax
import jax.numpy as jnp
from jax import lax
from jax.experimental import pallas as pl
from jax.experimental.pallas import tpu as pltpu
from jax.experimental.pallas import tpu_sc as plsc

N = 50000
E = 800000
H = 64
HH = 32
LAYERS = 6
EPS_LN = 1e-5

NTILES = 16
K = 256
GPC = K // 128
CHUNKS = 196
EP = NTILES * CHUNKS * K
NCH = NTILES * CHUNKS
AGG_ROWS = 50048
ROWS_PER_TILE = AGG_ROWS // NTILES

BN = 2000
NB = N // BN
BE = 8192
CNT = float(N * H)


def _sc_edge_body(h2, pack, weh, beh, agg_out,
                  pk0, pk1, msg0, msg1, wbuf,
                  agg_spmem, sg0, sg1, ss0, ss1):
    cid = lax.axis_index("c")
    sid = lax.axis_index("s")
    idx_bufs = (pk0, pk1)
    msg_bufs = (msg0, msg1)
    sgs = (sg0, sg1)
    sss = (ss0, ss1)

    pltpu.sync_copy(weh.at[0, cid], wbuf.at[0])
    pltpu.sync_copy(weh.at[1, cid], wbuf.at[1])
    pltpu.sync_copy(beh.at[cid], wbuf.at[2])
    w0a = wbuf[0, pl.ds(0, 16)]
    w0b = wbuf[0, pl.ds(16, 16)]
    w1a = wbuf[1, pl.ds(0, 16)]
    w1b = wbuf[1, pl.ds(16, 16)]
    bba = wbuf[2, pl.ds(0, 16)]
    bbb = wbuf[2, pl.ds(16, 16)]

    def _zero_row(r, carry):
        for rr in range(8):
            msg0[r * 8 + rr, pl.ds(0, 16)] = jnp.zeros((16,), jnp.float32)
            msg0[r * 8 + rr, pl.ds(16, 16)] = jnp.zeros((16,), jnp.float32)
        return carry
    lax.fori_loop(0, K // 8, _zero_row, 0)

    def _zero_copy(kk, carry):
        pltpu.sync_copy(
            msg0, agg_spmem.at[pl.ds(sid * ROWS_PER_TILE + kk * K, K)])
        return carry
    lax.fori_loop(0, ROWS_PER_TILE // K, _zero_copy, 0)
    pltpu.sync_copy(
        msg0.at[pl.ds(0, ROWS_PER_TILE % K)],
        agg_spmem.at[pl.ds(sid * ROWS_PER_TILE + (ROWS_PER_TILE // K) * K,
                           ROWS_PER_TILE % K)])
    plsc.subcore_barrier()

    def _g(t, b):
        ch = sid * CHUNKS + t
        mb, ib = msg_bufs[b], idx_bufs[b]

        @pl.when(t >= 2)
        def _():
            for j in range(GPC):
                pltpu.make_async_copy(
                    mb.at[pl.ds(j * 128, 128)], agg_spmem.at[ib.at[1, j]],
                    sss[b]).wait()

        pltpu.sync_copy(pack.at[cid, ch], ib)

        def _ecomp(i, carry):
            g = i // 8
            q = (i % 8) * 16
            va0 = plsc.bitcast(ib[2, g, pl.ds(q, 16)], jnp.float32)
            va1 = plsc.bitcast(ib[3, g, pl.ds(q, 16)], jnp.float32)
            for ii in range(16):
                a0 = jnp.broadcast_to(va0[ii], (16,))
                a1 = jnp.broadcast_to(va1[ii], (16,))
                mb[i * 16 + ii, pl.ds(0, 16)] = a0 * w0a + a1 * w1a + bba
                mb[i * 16 + ii, pl.ds(16, 16)] = a0 * w0b + a1 * w1b + bbb
            return carry
        lax.fori_loop(0, K // 16, _ecomp, 0)
        for j in range(GPC):
            pltpu.async_copy(h2.at[ib.at[0, j]],
                             mb.at[pl.ds(j * 128, 128)], sgs[b], add=True)

    def _r(t, b):
        mb, ib = msg_bufs[b], idx_bufs[b]
        for j in range(GPC):
            pltpu.make_async_copy(
                h2.at[ib.at[0, j]], mb.at[pl.ds(j * 128, 128)],
                sgs[b]).wait()

        def _relu(r, carry):
            for rr in range(8):
                for q in (0, 16):
                    v = mb[r * 8 + rr, pl.ds(q, 16)]
                    mb[r * 8 + rr, pl.ds(q, 16)] = jnp.maximum(v, 0.0)
            return carry
        lax.fori_loop(0, K // 8, _relu, 0)
        for j in range(GPC):
            pltpu.async_copy(mb.at[pl.ds(j * 128, 128)],
                             agg_spmem.at[ib.at[1, j]], sss[b], add=True)

    def _pair(g, carry):
        t0 = 2 * g
        _g(t0, 0)
        _g(t0 + 1, 1)
        _r(t0, 0)
        _r(t0 + 1, 1)
        return carry
    lax.fori_loop(0, CHUNKS // 2, _pair, 0)
    for b in range(2):
        for j in range(GPC):
            pltpu.make_async_copy(
                msg_bufs[b].at[pl.ds(j * 128, 128)],
                agg_spmem.at[idx_bufs[b].at[1, j]], sss[b]).wait()
    plsc.subcore_barrier()

    pltpu.sync_copy(
        agg_spmem.at[pl.ds(sid * ROWS_PER_TILE, ROWS_PER_TILE)],
        agg_out.at[pl.ds(sid * ROWS_PER_TILE, ROWS_PER_TILE), cid])


@functools.lru_cache(maxsize=None)
def _get_sc_edge():
    return pl.kernel(
        _sc_edge_body,
        out_type=jax.ShapeDtypeStruct((AGG_ROWS, 2, HH), jnp.float32),
        mesh=plsc.VectorSubcoreMesh(core_axis_name="c", subcore_axis_name="s"),
        scratch_types=[
            pltpu.VMEM((4, GPC, 128), jnp.int32),
            pltpu.VMEM((4, GPC, 128), jnp.int32),
            pltpu.VMEM((K, HH), jnp.float32),
            pltpu.VMEM((K, HH), jnp.float32),
            pltpu.VMEM((3, HH), jnp.float32),
            pltpu.VMEM_SHARED((AGG_ROWS, HH), jnp.float32),
            pltpu.SemaphoreType.DMA,
            pltpu.SemaphoreType.DMA,
            pltpu.SemaphoreType.DMA,
            pltpu.SemaphoreType.DMA,
        ],
        compiler_params=pltpu.CompilerParams(use_tc_tiling_on_sc=False,
                                             needs_layout_passes=False),
    )


NZ = N // 2
AGGZ = AGG_ROWS // 2


def _proj_nodes_body(x2_ref, wnd_ref, bnd_ref, hs_ref):
    x2 = x2_ref[...]
    hs_ref[...] = (x2[:, 0:1] * wnd_ref[0:1, :]
                   + x2[:, 1:2] * wnd_ref[1:2, :] + bnd_ref[...])


def _mlp_body(hs_ref, agg_ref, w1_ref, b1_ref, w2_ref, b2_ref,
              z_ref, ssum_ref, ssq_ref):
    zin = hs_ref[...] + agg_ref[...]
    t = jnp.maximum(
        jnp.dot(zin, w1_ref[...], preferred_element_type=jnp.float32,
                precision=lax.Precision.HIGHEST) + b1_ref[...], 0.0)
    z = jnp.dot(t, w2_ref[...], preferred_element_type=jnp.float32,
                precision=lax.Precision.HIGHEST) + b2_ref[...]
    z_ref[...] = z

    @pl.when(pl.program_id(0) == 0)
    def _():
        ssum_ref[0, 0] = 0.0
        ssq_ref[0, 0] = 0.0

    ssum_ref[0, 0] += jnp.sum(z)
    ssq_ref[0, 0] += jnp.sum(z * z)


def _norm_body(z_ref, hs_ref, ssum_ref, ssq_ref, g_ref, b_ref, out_ref):
    mu = ssum_ref[0, 0] / CNT
    var = jnp.maximum(ssq_ref[0, 0] / CNT - mu * mu, 0.0)
    rstd = 1.0 / (jnp.sqrt(var) + EPS_LN)
    z = (z_ref[...] - mu) * rstd * g_ref[...] + b_ref[...]
    out_ref[...] = jnp.maximum(z, 0.0) + hs_ref[...]


BZ = 1000
NBZ = NZ // BZ

_zspec = pl.BlockSpec((BZ, 128), lambda i: (i, 0))
_scal_smem = pl.BlockSpec((1, 1), lambda i: (0, 0), memory_space=pltpu.SMEM)


def _wfull(shape):
    return pl.BlockSpec(shape, lambda i: tuple(0 for _ in shape))


_proj_nodes = pl.pallas_call(
    _proj_nodes_body,
    grid=(NBZ,),
    in_specs=[pl.BlockSpec((BZ, 2), lambda i: (i, 0)),
              _wfull((2, 128)), _wfull((1, 128))],
    out_specs=_zspec,
    out_shape=jax.ShapeDtypeStruct((NZ, 128), jnp.float32),
)

_mlp = pl.pallas_call(
    _mlp_body,
    grid=(NBZ,),
    in_specs=[_zspec, _zspec, _wfull((128, 128)), _wfull((1, 128)),
              _wfull((128, 128)), _wfull((1, 128))],
    out_specs=(_zspec, _scal_smem, _scal_smem),
    out_shape=(jax.ShapeDtypeStruct((NZ, 128), jnp.float32),
               jax.ShapeDtypeStruct((1, 1), jnp.float32),
               jax.ShapeDtypeStruct((1, 1), jnp.float32)),
)

_norm = pl.pallas_call(
    _norm_body,
    grid=(NBZ,),
    in_specs=[_zspec, _zspec, _scal_smem, _scal_smem,
              _wfull((1, 128)), _wfull((1, 128))],
    out_specs=_zspec,
    out_shape=jax.ShapeDtypeStruct((NZ, 128), jnp.float32),
)


def kernel(x, edge_index, edge_attr, Wn, bn, We, be, W1, b1, W2, b2,
           gamma, beta):
    src = edge_index[0].astype(jnp.int32)
    dst = edge_index[1].astype(jnp.int32)

    pad = EP - E
    src_p = jnp.pad(src, (0, pad))
    dst_p = jnp.pad(dst, (0, pad), constant_values=N)
    src2 = jnp.stack([2 * src_p, 2 * src_p + 1])
    ea_p = jnp.pad(edge_attr, ((0, pad), (0, 0)))
    eab = lax.bitcast_convert_type(ea_p.T, jnp.int32)
    rest = jnp.broadcast_to(
        jnp.stack([dst_p, eab[0], eab[1]]), (2, 3, EP))
    pack = jnp.concatenate([src2[:, None], rest], axis=1)
    pack = pack.reshape(2, 4, NCH, GPC, 128).transpose(0, 2, 1, 3, 4)
    weh = We.reshape(2, 2, HH)
    beh = be.reshape(2, HH)

    def blockdiag2(w):
        zz = jnp.zeros((H, H), w.dtype)
        return jnp.concatenate(
            [jnp.concatenate([w, zz], axis=1),
             jnp.concatenate([zz, w], axis=1)], axis=0)

    wnd = jnp.concatenate(
        [jnp.concatenate([Wn, jnp.zeros((1, H), Wn.dtype)], axis=1),
         jnp.concatenate([jnp.zeros((1, H), Wn.dtype), Wn], axis=1)], axis=0)
    hs = _proj_nodes(x.reshape(NZ, 2), wnd, jnp.tile(bn, 2).reshape(1, 128))

    for l in range(LAYERS):
        h2 = hs.reshape(2 * N, HH)
        agg = _get_sc_edge()(h2, pack, weh, beh)
        aggz = agg.reshape(AGGZ, 128)
        z, ssum, ssq = _mlp(hs, aggz,
                            blockdiag2(W1[l]), jnp.tile(b1[l], 2).reshape(1, 128),
                            blockdiag2(W2[l]), jnp.tile(b2[l], 2).reshape(1, 128))
        hs = _norm(z, hs, ssum, ssq,
                   jnp.tile(gamma[l], 2).reshape(1, 128),
                   jnp.tile(beta[l], 2).reshape(1, 128))
    return hs.reshape(N, H)


# speedup vs baseline: 7.7490x; 1.2195x over previous
import functools

import jax
import jax.numpy as jnp
from jax import lax
from jax.experimental import pallas as pl
from jax.experimental.pallas import tpu as pltpu
from jax.experimental.pallas import tpu_sc as plsc

N = 50000
E = 800000
H = 64
HH = 32
LAYERS = 6
EPS_LN = 1e-5

NTILES = 16
K = 256
GPC = K // 128
CHUNKS = 196
EP = NTILES * CHUNKS * K
NCH = NTILES * CHUNKS
AGG_ROWS = 50048
ROWS_PER_TILE = AGG_ROWS // NTILES

BN = 2000
NB = N // BN
BE = 8192
CNT = float(N * H)


def _sc_edge_body(h2, pack, weh, beh, agg_out,
                  pk0, pk1, pk2, pk3, msg0, msg1, wbuf,
                  agg_spmem, sg0, sg1, ss0, ss1, sp0, sp1, sp2, sp3):
    cid = lax.axis_index("c")
    sid = lax.axis_index("s")
    pks = (pk0, pk1, pk2, pk3)
    sps = (sp0, sp1, sp2, sp3)
    msg_bufs = (msg0, msg1)
    sgs = (sg0, sg1)
    sss = (ss0, ss1)

    pltpu.sync_copy(weh.at[0, cid], wbuf.at[0])
    pltpu.sync_copy(weh.at[1, cid], wbuf.at[1])
    pltpu.sync_copy(beh.at[cid], wbuf.at[2])
    w0a = wbuf[0, pl.ds(0, 16)]
    w0b = wbuf[0, pl.ds(16, 16)]
    w1a = wbuf[1, pl.ds(0, 16)]
    w1b = wbuf[1, pl.ds(16, 16)]
    bba = wbuf[2, pl.ds(0, 16)]
    bbb = wbuf[2, pl.ds(16, 16)]

    def _zero_row(r, carry):
        for rr in range(8):
            msg0[r * 8 + rr, pl.ds(0, 16)] = jnp.zeros((16,), jnp.float32)
            msg0[r * 8 + rr, pl.ds(16, 16)] = jnp.zeros((16,), jnp.float32)
        return carry
    lax.fori_loop(0, K // 8, _zero_row, 0)

    def _zero_copy(kk, carry):
        pltpu.sync_copy(
            msg0, agg_spmem.at[pl.ds(sid * ROWS_PER_TILE + kk * K, K)])
        return carry
    lax.fori_loop(0, ROWS_PER_TILE // K, _zero_copy, 0)
    pltpu.sync_copy(
        msg0.at[pl.ds(0, ROWS_PER_TILE % K)],
        agg_spmem.at[pl.ds(sid * ROWS_PER_TILE + (ROWS_PER_TILE // K) * K,
                           ROWS_PER_TILE % K)])
    plsc.subcore_barrier()

    def _g(t, b, q):
        ch = sid * CHUNKS + t
        mb, ib = msg_bufs[b], pks[q]
        qn = (q + 2) % 4

        @pl.when(t >= 2)
        def _():
            for j in range(GPC):
                pltpu.make_async_copy(
                    mb.at[pl.ds(j * 128, 128)], agg_spmem.at[pks[qn].at[1, j]],
                    sss[b]).wait()

        @pl.when(t < CHUNKS - 2)
        def _():
            pltpu.async_copy(pack.at[cid, ch + 2], pks[qn], sps[qn])

        pltpu.make_async_copy(pack.at[cid, ch], ib, sps[q]).wait()

        def _ecomp(i, carry):
            g = i // 8
            q = (i % 8) * 16
            va0 = plsc.bitcast(ib[2, g, pl.ds(q, 16)], jnp.float32)
            va1 = plsc.bitcast(ib[3, g, pl.ds(q, 16)], jnp.float32)
            for ii in range(16):
                a0 = jnp.broadcast_to(va0[ii], (16,))
                a1 = jnp.broadcast_to(va1[ii], (16,))
                mb[i * 16 + ii, pl.ds(0, 16)] = a0 * w0a + a1 * w1a + bba
                mb[i * 16 + ii, pl.ds(16, 16)] = a0 * w0b + a1 * w1b + bbb
            return carry
        lax.fori_loop(0, K // 16, _ecomp, 0)
        for j in range(GPC):
            pltpu.async_copy(h2.at[ib.at[0, j]],
                             mb.at[pl.ds(j * 128, 128)], sgs[b], add=True)

    def _r(t, b, q):
        mb, ib = msg_bufs[b], pks[q]
        for j in range(GPC):
            pltpu.make_async_copy(
                h2.at[ib.at[0, j]], mb.at[pl.ds(j * 128, 128)],
                sgs[b]).wait()

        def _relu(r, carry):
            for rr in range(8):
                for q in (0, 16):
                    v = mb[r * 8 + rr, pl.ds(q, 16)]
                    mb[r * 8 + rr, pl.ds(q, 16)] = jnp.maximum(v, 0.0)
            return carry
        lax.fori_loop(0, K // 8, _relu, 0)
        for j in range(GPC):
            pltpu.async_copy(mb.at[pl.ds(j * 128, 128)],
                             agg_spmem.at[ib.at[1, j]], sss[b], add=True)

    pltpu.async_copy(pack.at[cid, sid * CHUNKS], pk0, sp0)
    pltpu.async_copy(pack.at[cid, sid * CHUNKS + 1], pk1, sp1)

    def _quad(g, carry):
        t0 = 4 * g
        for k in range(4):
            _g(t0 + k, k % 2, k)
            if k % 2 == 1:
                _r(t0 + k - 1, 0, k - 1)
                _r(t0 + k, 1, k)
        return carry
    lax.fori_loop(0, CHUNKS // 4, _quad, 0)
    for b in range(2):
        q = (CHUNKS - 2 + b) % 4
        for j in range(GPC):
            pltpu.make_async_copy(
                msg_bufs[b].at[pl.ds(j * 128, 128)],
                agg_spmem.at[pks[q].at[1, j]], sss[b]).wait()
    plsc.subcore_barrier()

    pltpu.sync_copy(
        agg_spmem.at[pl.ds(sid * ROWS_PER_TILE, ROWS_PER_TILE)],
        agg_out.at[pl.ds(sid * ROWS_PER_TILE, ROWS_PER_TILE), cid])


@functools.lru_cache(maxsize=None)
def _get_sc_edge():
    return pl.kernel(
        _sc_edge_body,
        out_type=jax.ShapeDtypeStruct((AGG_ROWS, 2, HH), jnp.float32),
        mesh=plsc.VectorSubcoreMesh(core_axis_name="c", subcore_axis_name="s"),
        scratch_types=[
            pltpu.VMEM((4, GPC, 128), jnp.int32),
            pltpu.VMEM((4, GPC, 128), jnp.int32),
            pltpu.VMEM((4, GPC, 128), jnp.int32),
            pltpu.VMEM((4, GPC, 128), jnp.int32),
            pltpu.VMEM((K, HH), jnp.float32),
            pltpu.VMEM((K, HH), jnp.float32),
            pltpu.VMEM((3, HH), jnp.float32),
            pltpu.VMEM_SHARED((AGG_ROWS, HH), jnp.float32),
            pltpu.SemaphoreType.DMA,
            pltpu.SemaphoreType.DMA,
            pltpu.SemaphoreType.DMA,
            pltpu.SemaphoreType.DMA,
            pltpu.SemaphoreType.DMA,
            pltpu.SemaphoreType.DMA,
            pltpu.SemaphoreType.DMA,
            pltpu.SemaphoreType.DMA,
        ],
        compiler_params=pltpu.CompilerParams(use_tc_tiling_on_sc=False,
                                             needs_layout_passes=False),
    )


NZ = N // 2
AGGZ = AGG_ROWS // 2


def _proj_nodes_body(x2_ref, wnd_ref, bnd_ref, hs_ref):
    x2 = x2_ref[...]
    hs_ref[...] = (x2[:, 0:1] * wnd_ref[0:1, :]
                   + x2[:, 1:2] * wnd_ref[1:2, :] + bnd_ref[...])


def _mlp_body(hs_ref, agg_ref, w1_ref, b1_ref, w2_ref, b2_ref,
              z_ref, ssum_ref, ssq_ref):
    zin = hs_ref[...] + agg_ref[...]
    t = jnp.maximum(
        jnp.dot(zin, w1_ref[...], preferred_element_type=jnp.float32,
                precision=lax.Precision.HIGHEST) + b1_ref[...], 0.0)
    z = jnp.dot(t, w2_ref[...], preferred_element_type=jnp.float32,
                precision=lax.Precision.HIGHEST) + b2_ref[...]
    z_ref[...] = z

    @pl.when(pl.program_id(0) == 0)
    def _():
        ssum_ref[0, 0] = 0.0
        ssq_ref[0, 0] = 0.0

    ssum_ref[0, 0] += jnp.sum(z)
    ssq_ref[0, 0] += jnp.sum(z * z)


def _norm_body(z_ref, hs_ref, ssum_ref, ssq_ref, g_ref, b_ref, out_ref):
    mu = ssum_ref[0, 0] / CNT
    var = jnp.maximum(ssq_ref[0, 0] / CNT - mu * mu, 0.0)
    rstd = 1.0 / (jnp.sqrt(var) + EPS_LN)
    z = (z_ref[...] - mu) * rstd * g_ref[...] + b_ref[...]
    out_ref[...] = jnp.maximum(z, 0.0) + hs_ref[...]


BZ = 1000
NBZ = NZ // BZ

_zspec = pl.BlockSpec((BZ, 128), lambda i: (i, 0))
_scal_smem = pl.BlockSpec((1, 1), lambda i: (0, 0), memory_space=pltpu.SMEM)


def _wfull(shape):
    return pl.BlockSpec(shape, lambda i: tuple(0 for _ in shape))


_proj_nodes = pl.pallas_call(
    _proj_nodes_body,
    grid=(NBZ,),
    in_specs=[pl.BlockSpec((BZ, 2), lambda i: (i, 0)),
              _wfull((2, 128)), _wfull((1, 128))],
    out_specs=_zspec,
    out_shape=jax.ShapeDtypeStruct((NZ, 128), jnp.float32),
)

_mlp = pl.pallas_call(
    _mlp_body,
    grid=(NBZ,),
    in_specs=[_zspec, _zspec, _wfull((128, 128)), _wfull((1, 128)),
              _wfull((128, 128)), _wfull((1, 128))],
    out_specs=(_zspec, _scal_smem, _scal_smem),
    out_shape=(jax.ShapeDtypeStruct((NZ, 128), jnp.float32),
               jax.ShapeDtypeStruct((1, 1), jnp.float32),
               jax.ShapeDtypeStruct((1, 1), jnp.float32)),
)

_norm = pl.pallas_call(
    _norm_body,
    grid=(NBZ,),
    in_specs=[_zspec, _zspec, _scal_smem, _scal_smem,
              _wfull((1, 128)), _wfull((1, 128))],
    out_specs=_zspec,
    out_shape=jax.ShapeDtypeStruct((NZ, 128), jnp.float32),
)


def kernel(x, edge_index, edge_attr, Wn, bn, We, be, W1, b1, W2, b2,
           gamma, beta):
    src = edge_index[0].astype(jnp.int32)
    dst = edge_index[1].astype(jnp.int32)

    pad = EP - E
    src_p = jnp.pad(src, (0, pad))
    dst_p = jnp.pad(dst, (0, pad), constant_values=N)
    src2 = jnp.stack([2 * src_p, 2 * src_p + 1])
    ea_p = jnp.pad(edge_attr, ((0, pad), (0, 0)))
    eab = lax.bitcast_convert_type(ea_p.T, jnp.int32)
    rest = jnp.broadcast_to(
        jnp.stack([dst_p, eab[0], eab[1]]), (2, 3, EP))
    pack = jnp.concatenate([src2[:, None], rest], axis=1)
    pack = pack.reshape(2, 4, NCH, GPC, 128).transpose(0, 2, 1, 3, 4)
    weh = We.reshape(2, 2, HH)
    beh = be.reshape(2, HH)

    def blockdiag2(w):
        zz = jnp.zeros((H, H), w.dtype)
        return jnp.concatenate(
            [jnp.concatenate([w, zz], axis=1),
             jnp.concatenate([zz, w], axis=1)], axis=0)

    wnd = jnp.concatenate(
        [jnp.concatenate([Wn, jnp.zeros((1, H), Wn.dtype)], axis=1),
         jnp.concatenate([jnp.zeros((1, H), Wn.dtype), Wn], axis=1)], axis=0)
    hs = _proj_nodes(x.reshape(NZ, 2), wnd, jnp.tile(bn, 2).reshape(1, 128))

    for l in range(LAYERS):
        h2 = hs.reshape(2 * N, HH)
        agg = _get_sc_edge()(h2, pack, weh, beh)
        aggz = agg.reshape(AGGZ, 128)
        z, ssum, ssq = _mlp(hs, aggz,
                            blockdiag2(W1[l]), jnp.tile(b1[l], 2).reshape(1, 128),
                            blockdiag2(W2[l]), jnp.tile(b2[l], 2).reshape(1, 128))
        hs = _norm(z, hs, ssum, ssq,
                   jnp.tile(gamma[l], 2).reshape(1, 128),
                   jnp.tile(beta[l], 2).reshape(1, 128))
    return hs.reshape(N, H)
